# Initial kernel scaffold; baseline (speedup 1.0000x reference)
import jax
import jax.numpy as jnp
from jax.experimental import pallas as pl


def kernel(x, pos, batch, edge_index, edge_weights, table, W_in, b_in, msg_W1, msg_b1, msg_W2, msg_b2, upd_W1, upd_b1, upd_W2, upd_b2, W_out1, b_out1, W_out2, b_out2):
    raise NotImplementedError("write your pallas kernel here")


# speedup vs baseline: 1.0002x; 1.0002x over previous
import jax
import jax.numpy as jnp
from jax.experimental import pallas as pl

N = 100000
E = 1600000
HIDDEN = 128
DEPTH = 3
G = 16
SH = 9


def _sh(r):
    x, y, z = r[:, 0], r[:, 1], r[:, 2]
    sh0 = jnp.ones_like(x)[:, None]
    sh1 = jnp.sqrt(3.0) * r
    sh2 = jnp.stack([
        jnp.sqrt(15.0) * x * y,
        jnp.sqrt(15.0) * y * z,
        (jnp.sqrt(5.0) / 2.0) * (2.0 * z * z - x * x - y * y),
        jnp.sqrt(15.0) * x * z,
        (jnp.sqrt(15.0) / 2.0) * (x * x - y * y),
    ], axis=-1)
    return jnp.concatenate([sh0, sh1, sh2], axis=-1)


def kernel(x, pos, batch, edge_index, edge_weights, table, W_in, b_in,
           msg_W1, msg_b1, msg_W2, msg_b2, upd_W1, upd_b1, upd_W2, upd_b2,
           W_out1, b_out1, W_out2, b_out2):
    swish = jax.nn.silu
    row, col = edge_index[0], edge_index[1]
    x_emb = jnp.take(table, x, axis=0)
    rel = pos[row] - pos[col]
    rel = rel / jnp.maximum(jnp.linalg.norm(rel, axis=-1, keepdims=True), 1e-6)
    edge_attr = _sh(rel)
    esum = jax.ops.segment_sum(edge_attr, col, num_segments=N)
    ecnt = jax.ops.segment_sum(jnp.ones((E,), jnp.float32), col, num_segments=N)
    node_attr = esum / jnp.maximum(ecnt, 1.0)[:, None]
    h = swish(jnp.concatenate([x_emb, node_attr], axis=-1) @ W_in + b_in)
    d = edge_weights[:, None]
    counts = jax.ops.segment_sum(jnp.ones((N,), jnp.float32), batch, num_segments=G)
    for l in range(DEPTH):
        m_in = jnp.concatenate([h[row], h[col], edge_attr, d], axis=-1)
        m = swish(m_in @ msg_W1[l] + msg_b1[l])
        m = swish(m @ msg_W2[l] + msg_b2[l])
        agg = jax.ops.segment_sum(m, col, num_segments=N)
        u_in = jnp.concatenate([h, agg, node_attr], axis=-1)
        u = swish(u_in @ upd_W1[l] + upd_b1[l])
        u = u @ upd_W2[l] + upd_b2[l]
        h = h + u
        mean = jax.ops.segment_sum(h, batch, num_segments=G) / counts[:, None]
        var = jax.ops.segment_sum(h * h, batch, num_segments=G) / counts[:, None] - mean ** 2
        h = (h - mean[batch]) / jnp.sqrt(jnp.maximum(var[batch], 0.0) + 1e-5)
    pooled = jax.ops.segment_sum(h, batch, num_segments=G) / counts[:, None]
    out = swish(pooled @ W_out1 + b_out1) @ W_out2 + b_out2
    return out


# speedup vs baseline: 1.3632x; 1.3629x over previous
import functools

import jax
import jax.numpy as jnp
from jax import lax
from jax.experimental import pallas as pl
from jax.experimental.pallas import tpu as pltpu
from jax.experimental.pallas import tpu_sc as plsc

N = 100000
E = 1600000
HIDDEN = 128
DEPTH = 3
G = 16
SH = 9

_NC = 2
_NS = 16
_NW = _NC * _NS

_PW = E // _NW
_CH = 128
_NFULL = _PW // _CH
_TAIL = _PW - _NFULL * _CH


def _edge_gather(h, rowi, coli):
    mesh = plsc.VectorSubcoreMesh(core_axis_name="c", subcore_axis_name="s")

    @functools.partial(
        pl.kernel, mesh=mesh,
        out_type=(jax.ShapeDtypeStruct((E, HIDDEN), jnp.float32),
                  jax.ShapeDtypeStruct((E, HIDDEN), jnp.float32)),
        scratch_types=[
            pltpu.VMEM((_PW,), jnp.int32),
            pltpu.VMEM((_CH, HIDDEN), jnp.float32),
            pltpu.VMEM((_CH, HIDDEN), jnp.float32),
            pltpu.SemaphoreType.DMA,
            pltpu.SemaphoreType.DMA,
        ],
    )
    def gk(tbl, ridx, cidx, out_r, out_c, idx_v, buf0, buf1, sem0, sem1):
        wid = lax.axis_index("s") * _NC + lax.axis_index("c")
        base = wid * _PW
        for idx_hbm, out_hbm in ((ridx, out_r), (cidx, out_c)):
            pltpu.sync_copy(idx_hbm.at[pl.ds(base, _PW)], idx_v)

            def body(i, _):
                off = i * (2 * _CH)
                pltpu.async_copy(tbl.at[idx_v.at[pl.ds(off, _CH)]], buf0, sem0)
                pltpu.async_copy(tbl.at[idx_v.at[pl.ds(off + _CH, _CH)]], buf1, sem1)
                pltpu.make_async_copy(tbl.at[idx_v.at[pl.ds(off, _CH)]], buf0, sem0).wait()
                pltpu.sync_copy(buf0, out_hbm.at[pl.ds(base + off, _CH)])
                pltpu.make_async_copy(tbl.at[idx_v.at[pl.ds(off + _CH, _CH)]], buf1, sem1).wait()
                pltpu.sync_copy(buf1, out_hbm.at[pl.ds(base + off + _CH, _CH)])
                return _

            lax.fori_loop(0, _NFULL // 2, body, 0)
            toff = _NFULL * _CH
            pltpu.async_copy(tbl.at[idx_v.at[pl.ds(toff, _TAIL)]],
                             buf0.at[pl.ds(0, _TAIL)], sem0).wait()
            pltpu.sync_copy(buf0.at[pl.ds(0, _TAIL)],
                            out_hbm.at[pl.ds(base + toff, _TAIL)])

    return gk(h, rowi, coli)


_BE = 2000
_BN = 2000


def _msg_body(hr, hc, ead, w1, b1, w2, b2, o, acc):
    m_in = jnp.concatenate([hr[...], hc[...], ead[...]], axis=-1)
    acc[...] = jnp.dot(m_in, w1[...], preferred_element_type=jnp.float32)
    z = jax.nn.silu(acc[...] + b1[...])
    acc[...] = jnp.dot(z, w2[...], preferred_element_type=jnp.float32)
    o[...] = jax.nn.silu(acc[...] + b2[...])


def _msg_mlp(hr, hc, ead, w1, b1, w2, b2):
    grid = (E // _BE,)
    blk = lambda r, c: pl.BlockSpec((r, c), lambda i: (i, 0))
    fix = lambda r, c: pl.BlockSpec((r, c), lambda i: (0, 0))
    return pl.pallas_call(
        _msg_body,
        grid=grid,
        in_specs=[blk(_BE, HIDDEN), blk(_BE, HIDDEN), blk(_BE, SH + 1),
                  fix(2 * HIDDEN + SH + 1, HIDDEN), fix(1, HIDDEN),
                  fix(HIDDEN, HIDDEN), fix(1, HIDDEN)],
        out_specs=blk(_BE, HIDDEN),
        out_shape=jax.ShapeDtypeStruct((E, HIDDEN), jnp.float32),
        scratch_shapes=[pltpu.VMEM((_BE, HIDDEN), jnp.float32)],
    )(hr, hc, ead, w1, b1.reshape(1, HIDDEN), w2, b2.reshape(1, HIDDEN))


def _upd_body(h, ag, na, u1, b1, u2, b2, o, acc):
    u_in = jnp.concatenate([h[...], ag[...], na[...]], axis=-1)
    acc[...] = jnp.dot(u_in, u1[...], preferred_element_type=jnp.float32)
    z = jax.nn.silu(acc[...] + b1[...])
    acc[...] = jnp.dot(z, u2[...], preferred_element_type=jnp.float32)
    o[...] = h[...] + (acc[...] + b2[...])


def _upd_mlp(h, agg, na, u1, b1, u2, b2):
    grid = (N // _BN,)
    blk = lambda r, c: pl.BlockSpec((r, c), lambda i: (i, 0))
    fix = lambda r, c: pl.BlockSpec((r, c), lambda i: (0, 0))
    return pl.pallas_call(
        _upd_body,
        grid=grid,
        in_specs=[blk(_BN, HIDDEN), blk(_BN, HIDDEN), blk(_BN, SH),
                  fix(2 * HIDDEN + SH, HIDDEN), fix(1, HIDDEN),
                  fix(HIDDEN, HIDDEN), fix(1, HIDDEN)],
        out_specs=blk(_BN, HIDDEN),
        out_shape=jax.ShapeDtypeStruct((N, HIDDEN), jnp.float32),
        scratch_shapes=[pltpu.VMEM((_BN, HIDDEN), jnp.float32)],
    )(h, agg, na, u1, b1.reshape(1, HIDDEN), u2, b2.reshape(1, HIDDEN))


def _sh(r):
    x, y, z = r[:, 0], r[:, 1], r[:, 2]
    sh0 = jnp.ones_like(x)[:, None]
    sh1 = jnp.sqrt(3.0) * r
    sh2 = jnp.stack([
        jnp.sqrt(15.0) * x * y,
        jnp.sqrt(15.0) * y * z,
        (jnp.sqrt(5.0) / 2.0) * (2.0 * z * z - x * x - y * y),
        jnp.sqrt(15.0) * x * z,
        (jnp.sqrt(15.0) / 2.0) * (x * x - y * y),
    ], axis=-1)
    return jnp.concatenate([sh0, sh1, sh2], axis=-1)


def kernel(x, pos, batch, edge_index, edge_weights, table, W_in, b_in,
           msg_W1, msg_b1, msg_W2, msg_b2, upd_W1, upd_b1, upd_W2, upd_b2,
           W_out1, b_out1, W_out2, b_out2):
    swish = jax.nn.silu
    row, col = edge_index[0], edge_index[1]
    x_emb = jnp.take(table, x, axis=0)
    rel = pos[row] - pos[col]
    rel = rel / jnp.maximum(jnp.linalg.norm(rel, axis=-1, keepdims=True), 1e-6)
    edge_attr = _sh(rel)
    esum = jax.ops.segment_sum(edge_attr, col, num_segments=N)
    ecnt = jax.ops.segment_sum(jnp.ones((E,), jnp.float32), col, num_segments=N)
    node_attr = esum / jnp.maximum(ecnt, 1.0)[:, None]
    h = swish(jnp.concatenate([x_emb, node_attr], axis=-1) @ W_in + b_in)
    d = edge_weights[:, None]
    ead = jnp.concatenate([edge_attr, d], axis=-1)
    counts = jax.ops.segment_sum(jnp.ones((N,), jnp.float32), batch, num_segments=G)
    for l in range(DEPTH):
        hr, hc = _edge_gather(h, row, col)
        m = _msg_mlp(hr, hc, ead, msg_W1[l], msg_b1[l], msg_W2[l], msg_b2[l])
        agg = jax.ops.segment_sum(m, col, num_segments=N)
        h = _upd_mlp(h, agg, node_attr, upd_W1[l], upd_b1[l], upd_W2[l], upd_b2[l])
        mean = jax.ops.segment_sum(h, batch, num_segments=G) / counts[:, None]
        var = jax.ops.segment_sum(h * h, batch, num_segments=G) / counts[:, None] - mean ** 2
        h = (h - mean[batch]) / jnp.sqrt(jnp.maximum(var[batch], 0.0) + 1e-5)
    pooled = jax.ops.segment_sum(h, batch, num_segments=G) / counts[:, None]
    out = swish(pooled @ W_out1 + b_out1) @ W_out2 + b_out2
    return out


# speedup vs baseline: 1.5907x; 1.1668x over previous
import functools

import jax
import jax.numpy as jnp
from jax import lax
from jax.experimental import pallas as pl
from jax.experimental.pallas import tpu as pltpu
from jax.experimental.pallas import tpu_sc as plsc

N = 100000
E = 1600000
HIDDEN = 128
DEPTH = 3
G = 16
SH = 9

_NC = 2
_NS = 16
_NW = _NC * _NS

_PW = E // _NW
_CH = 128
_NFULL = _PW // _CH
_TAIL = _PW - _NFULL * _CH


def _edge_gather(h, rowi, coli):
    mesh = plsc.VectorSubcoreMesh(core_axis_name="c", subcore_axis_name="s")

    @functools.partial(
        pl.kernel, mesh=mesh,
        out_type=(jax.ShapeDtypeStruct((E, HIDDEN), jnp.float32),
                  jax.ShapeDtypeStruct((E, HIDDEN), jnp.float32)),
        scratch_types=[
            pltpu.VMEM((_PW,), jnp.int32),
            pltpu.VMEM((_CH, HIDDEN), jnp.float32),
            pltpu.VMEM((_CH, HIDDEN), jnp.float32),
            pltpu.SemaphoreType.DMA,
            pltpu.SemaphoreType.DMA,
        ],
    )
    def gk(tbl, ridx, cidx, out_r, out_c, idx_v, buf0, buf1, sem0, sem1):
        wid = lax.axis_index("s") * _NC + lax.axis_index("c")
        base = wid * _PW
        for idx_hbm, out_hbm in ((ridx, out_r), (cidx, out_c)):
            pltpu.sync_copy(idx_hbm.at[pl.ds(base, _PW)], idx_v)

            def body(i, _):
                off = i * (2 * _CH)
                pltpu.async_copy(tbl.at[idx_v.at[pl.ds(off, _CH)]], buf0, sem0)
                pltpu.async_copy(tbl.at[idx_v.at[pl.ds(off + _CH, _CH)]], buf1, sem1)
                pltpu.make_async_copy(tbl.at[idx_v.at[pl.ds(off, _CH)]], buf0, sem0).wait()
                pltpu.sync_copy(buf0, out_hbm.at[pl.ds(base + off, _CH)])
                pltpu.make_async_copy(tbl.at[idx_v.at[pl.ds(off + _CH, _CH)]], buf1, sem1).wait()
                pltpu.sync_copy(buf1, out_hbm.at[pl.ds(base + off + _CH, _CH)])
                return _

            lax.fori_loop(0, _NFULL // 2, body, 0)
            toff = _NFULL * _CH
            pltpu.async_copy(tbl.at[idx_v.at[pl.ds(toff, _TAIL)]],
                             buf0.at[pl.ds(0, _TAIL)], sem0).wait()
            pltpu.sync_copy(buf0.at[pl.ds(0, _TAIL)],
                            out_hbm.at[pl.ds(base + toff, _TAIL)])

    return gk(h, rowi, coli)


_BE = 2000
_BN = 2000


def _msg_body(hr, hc, ead, w1, b1, w2, b2, o, acc):
    m_in = jnp.concatenate([hr[...], hc[...], ead[...]], axis=-1)
    acc[...] = jnp.dot(m_in, w1[...], preferred_element_type=jnp.float32)
    z = jax.nn.silu(acc[...] + b1[...])
    acc[...] = jnp.dot(z, w2[...], preferred_element_type=jnp.float32)
    o[...] = jax.nn.silu(acc[...] + b2[...])


def _msg_mlp(hr, hc, ead, w1, b1, w2, b2):
    grid = (E // _BE,)
    blk = lambda r, c: pl.BlockSpec((r, c), lambda i: (i, 0))
    fix = lambda r, c: pl.BlockSpec((r, c), lambda i: (0, 0))
    return pl.pallas_call(
        _msg_body,
        grid=grid,
        in_specs=[blk(_BE, HIDDEN), blk(_BE, HIDDEN), blk(_BE, SH + 1),
                  fix(2 * HIDDEN + SH + 1, HIDDEN), fix(1, HIDDEN),
                  fix(HIDDEN, HIDDEN), fix(1, HIDDEN)],
        out_specs=blk(_BE, HIDDEN),
        out_shape=jax.ShapeDtypeStruct((E, HIDDEN), jnp.float32),
        scratch_shapes=[pltpu.VMEM((_BE, HIDDEN), jnp.float32)],
    )(hr, hc, ead, w1, b1.reshape(1, HIDDEN), w2, b2.reshape(1, HIDDEN))


def _upd_body(h, ag, na, u1, b1, u2, b2, o, acc):
    u_in = jnp.concatenate([h[...], ag[...], na[...]], axis=-1)
    acc[...] = jnp.dot(u_in, u1[...], preferred_element_type=jnp.float32)
    z = jax.nn.silu(acc[...] + b1[...])
    acc[...] = jnp.dot(z, u2[...], preferred_element_type=jnp.float32)
    o[...] = h[...] + (acc[...] + b2[...])


def _upd_mlp(h, agg, na, u1, b1, u2, b2):
    grid = (N // _BN,)
    blk = lambda r, c: pl.BlockSpec((r, c), lambda i: (i, 0))
    fix = lambda r, c: pl.BlockSpec((r, c), lambda i: (0, 0))
    return pl.pallas_call(
        _upd_body,
        grid=grid,
        in_specs=[blk(_BN, HIDDEN), blk(_BN, HIDDEN), blk(_BN, SH),
                  fix(2 * HIDDEN + SH, HIDDEN), fix(1, HIDDEN),
                  fix(HIDDEN, HIDDEN), fix(1, HIDDEN)],
        out_specs=blk(_BN, HIDDEN),
        out_shape=jax.ShapeDtypeStruct((N, HIDDEN), jnp.float32),
        scratch_shapes=[pltpu.VMEM((_BN, HIDDEN), jnp.float32)],
    )(h, agg, na, u1, b1.reshape(1, HIDDEN), u2, b2.reshape(1, HIDDEN))


def _sh(r):
    x, y, z = r[:, 0], r[:, 1], r[:, 2]
    sh0 = jnp.ones_like(x)[:, None]
    sh1 = jnp.sqrt(3.0) * r
    sh2 = jnp.stack([
        jnp.sqrt(15.0) * x * y,
        jnp.sqrt(15.0) * y * z,
        (jnp.sqrt(5.0) / 2.0) * (2.0 * z * z - x * x - y * y),
        jnp.sqrt(15.0) * x * z,
        (jnp.sqrt(15.0) / 2.0) * (x * x - y * y),
    ], axis=-1)
    return jnp.concatenate([sh0, sh1, sh2], axis=-1)


def kernel(x, pos, batch, edge_index, edge_weights, table, W_in, b_in,
           msg_W1, msg_b1, msg_W2, msg_b2, upd_W1, upd_b1, upd_W2, upd_b2,
           W_out1, b_out1, W_out2, b_out2):
    swish = jax.nn.silu
    row, col = edge_index[0], edge_index[1]
    x_emb = jnp.take(table, x, axis=0)
    posP = jnp.pad(pos, ((0, 0), (0, HIDDEN - 3)))
    prP, pcP = _edge_gather(posP, row, col)
    rel = prP[:, :3] - pcP[:, :3]
    rel = rel / jnp.maximum(jnp.linalg.norm(rel, axis=-1, keepdims=True), 1e-6)
    edge_attr = _sh(rel)
    esum = jax.ops.segment_sum(edge_attr, col, num_segments=N)
    ecnt = jax.ops.segment_sum(jnp.ones((E,), jnp.float32), col, num_segments=N)
    node_attr = esum / jnp.maximum(ecnt, 1.0)[:, None]
    h = swish(jnp.concatenate([x_emb, node_attr], axis=-1) @ W_in + b_in)
    d = edge_weights[:, None]
    ead = jnp.concatenate([edge_attr, d], axis=-1)
    counts = jax.ops.segment_sum(jnp.ones((N,), jnp.float32), batch, num_segments=G)
    for l in range(DEPTH):
        hr, hc = _edge_gather(h, row, col)
        m = _msg_mlp(hr, hc, ead, msg_W1[l], msg_b1[l], msg_W2[l], msg_b2[l])
        agg = jax.ops.segment_sum(m, col, num_segments=N)
        h = _upd_mlp(h, agg, node_attr, upd_W1[l], upd_b1[l], upd_W2[l], upd_b2[l])
        mean = jax.ops.segment_sum(h, batch, num_segments=G) / counts[:, None]
        var = jax.ops.segment_sum(h * h, batch, num_segments=G) / counts[:, None] - mean ** 2
        h = (h - mean[batch]) / jnp.sqrt(jnp.maximum(var[batch], 0.0) + 1e-5)
    pooled = jax.ops.segment_sum(h, batch, num_segments=G) / counts[:, None]
    out = swish(pooled @ W_out1 + b_out1) @ W_out2 + b_out2
    return out


# speedup vs baseline: 2.0737x; 1.3036x over previous
import functools

import jax
import jax.numpy as jnp
from jax import lax
from jax.experimental import pallas as pl
from jax.experimental.pallas import tpu as pltpu
from jax.experimental.pallas import tpu_sc as plsc

N = 100000
E = 1600000
HIDDEN = 128
DEPTH = 3
G = 16
SH = 9

_NC = 2
_NS = 16
_NW = _NC * _NS

_PW = E // _NW
_CH = 128
_NFULL = _PW // _CH
_TAIL = _PW - _NFULL * _CH


def _edge_gather(h, rowi, coli):
    mesh = plsc.VectorSubcoreMesh(core_axis_name="c", subcore_axis_name="s")

    @functools.partial(
        pl.kernel, mesh=mesh,
        out_type=(jax.ShapeDtypeStruct((E, HIDDEN), jnp.float32),
                  jax.ShapeDtypeStruct((E, HIDDEN), jnp.float32)),
        scratch_types=[
            pltpu.VMEM((_PW,), jnp.int32),
            pltpu.VMEM((_CH, HIDDEN), jnp.float32),
            pltpu.VMEM((_CH, HIDDEN), jnp.float32),
            pltpu.SemaphoreType.DMA,
            pltpu.SemaphoreType.DMA,
        ],
    )
    def gk(tbl, ridx, cidx, out_r, out_c, idx_v, buf0, buf1, sem0, sem1):
        wid = lax.axis_index("s") * _NC + lax.axis_index("c")
        base = wid * _PW
        for idx_hbm, out_hbm in ((ridx, out_r), (cidx, out_c)):
            pltpu.sync_copy(idx_hbm.at[pl.ds(base, _PW)], idx_v)

            def body(i, _):
                off = i * (2 * _CH)
                pltpu.async_copy(tbl.at[idx_v.at[pl.ds(off, _CH)]], buf0, sem0)
                pltpu.async_copy(tbl.at[idx_v.at[pl.ds(off + _CH, _CH)]], buf1, sem1)
                pltpu.make_async_copy(tbl.at[idx_v.at[pl.ds(off, _CH)]], buf0, sem0).wait()
                pltpu.sync_copy(buf0, out_hbm.at[pl.ds(base + off, _CH)])
                pltpu.make_async_copy(tbl.at[idx_v.at[pl.ds(off + _CH, _CH)]], buf1, sem1).wait()
                pltpu.sync_copy(buf1, out_hbm.at[pl.ds(base + off + _CH, _CH)])
                return _

            lax.fori_loop(0, _NFULL // 2, body, 0)
            toff = _NFULL * _CH
            pltpu.async_copy(tbl.at[idx_v.at[pl.ds(toff, _TAIL)]],
                             buf0.at[pl.ds(0, _TAIL)], sem0).wait()
            pltpu.sync_copy(buf0.at[pl.ds(0, _TAIL)],
                            out_hbm.at[pl.ds(base + toff, _TAIL)])

    return gk(h, rowi, coli)


_BE = 2000
_BN = 2000


def _eau_body(rn, dref, o):
    r = rn[...]
    x, y, z = r[:, 0:1], r[:, 1:2], r[:, 2:3]
    rel3 = r[:, 0:3]
    sh0 = jnp.ones_like(x)
    sh1 = jnp.sqrt(3.0) * rel3
    sh2 = jnp.concatenate([
        jnp.sqrt(15.0) * x * y,
        jnp.sqrt(15.0) * y * z,
        (jnp.sqrt(5.0) / 2.0) * (2.0 * z * z - x * x - y * y),
        jnp.sqrt(15.0) * x * z,
        (jnp.sqrt(15.0) / 2.0) * (x * x - y * y)], axis=-1)
    dcol = jnp.reshape(dref[...], (_BE, 1))
    o[...] = jnp.concatenate([sh0, sh1, sh2, dcol, sh0], axis=-1)


def _eau_kernel(relnP, edge_weights):
    return pl.pallas_call(
        _eau_body, grid=(E // _BE,),
        in_specs=[pl.BlockSpec((_BE, HIDDEN), lambda i: (i, 0)),
                  pl.BlockSpec((1, 1, _BE), lambda i: (i, 0, 0))],
        out_specs=pl.BlockSpec((_BE, SH + 2), lambda i: (i, 0)),
        out_shape=jax.ShapeDtypeStruct((E, SH + 2), jnp.float32),
    )(relnP, edge_weights.reshape(E // _BE, 1, _BE))


def _msg_body(hr, hc, eau, w1, b1, w2, b2, o, acc):
    m_in = jnp.concatenate([hr[...], hc[...], eau[...][:, :SH + 1]], axis=-1)
    acc[...] = jnp.dot(m_in, w1[...], preferred_element_type=jnp.float32)
    z = jax.nn.silu(acc[...] + b1[...])
    acc[...] = jnp.dot(z, w2[...], preferred_element_type=jnp.float32)
    o[...] = jax.nn.silu(acc[...] + b2[...])


def _msg_mlp(hr, hc, ead, w1, b1, w2, b2):
    grid = (E // _BE,)
    blk = lambda r, c: pl.BlockSpec((r, c), lambda i: (i, 0))
    fix = lambda r, c: pl.BlockSpec((r, c), lambda i: (0, 0))
    return pl.pallas_call(
        _msg_body,
        grid=grid,
        in_specs=[blk(_BE, HIDDEN), blk(_BE, HIDDEN), blk(_BE, SH + 2),
                  fix(2 * HIDDEN + SH + 1, HIDDEN), fix(1, HIDDEN),
                  fix(HIDDEN, HIDDEN), fix(1, HIDDEN)],
        out_specs=blk(_BE, HIDDEN),
        out_shape=jax.ShapeDtypeStruct((E, HIDDEN), jnp.float32),
        scratch_shapes=[pltpu.VMEM((_BE, HIDDEN), jnp.float32)],
    )(hr, hc, ead, w1, b1.reshape(1, HIDDEN), w2, b2.reshape(1, HIDDEN))


def _upd_body(h, ag, na, u1, b1, u2, b2, o, acc):
    u_in = jnp.concatenate([h[...], ag[...], na[...]], axis=-1)
    acc[...] = jnp.dot(u_in, u1[...], preferred_element_type=jnp.float32)
    z = jax.nn.silu(acc[...] + b1[...])
    acc[...] = jnp.dot(z, u2[...], preferred_element_type=jnp.float32)
    o[...] = h[...] + (acc[...] + b2[...])


def _upd_mlp(h, agg, na, u1, b1, u2, b2):
    grid = (N // _BN,)
    blk = lambda r, c: pl.BlockSpec((r, c), lambda i: (i, 0))
    fix = lambda r, c: pl.BlockSpec((r, c), lambda i: (0, 0))
    return pl.pallas_call(
        _upd_body,
        grid=grid,
        in_specs=[blk(_BN, HIDDEN), blk(_BN, HIDDEN), blk(_BN, SH),
                  fix(2 * HIDDEN + SH, HIDDEN), fix(1, HIDDEN),
                  fix(HIDDEN, HIDDEN), fix(1, HIDDEN)],
        out_specs=blk(_BN, HIDDEN),
        out_shape=jax.ShapeDtypeStruct((N, HIDDEN), jnp.float32),
        scratch_shapes=[pltpu.VMEM((_BN, HIDDEN), jnp.float32)],
    )(h, agg, na, u1, b1.reshape(1, HIDDEN), u2, b2.reshape(1, HIDDEN))


def _sh(r):
    x, y, z = r[:, 0], r[:, 1], r[:, 2]
    sh0 = jnp.ones_like(x)[:, None]
    sh1 = jnp.sqrt(3.0) * r
    sh2 = jnp.stack([
        jnp.sqrt(15.0) * x * y,
        jnp.sqrt(15.0) * y * z,
        (jnp.sqrt(5.0) / 2.0) * (2.0 * z * z - x * x - y * y),
        jnp.sqrt(15.0) * x * z,
        (jnp.sqrt(15.0) / 2.0) * (x * x - y * y),
    ], axis=-1)
    return jnp.concatenate([sh0, sh1, sh2], axis=-1)


def kernel(x, pos, batch, edge_index, edge_weights, table, W_in, b_in,
           msg_W1, msg_b1, msg_W2, msg_b2, upd_W1, upd_b1, upd_W2, upd_b2,
           W_out1, b_out1, W_out2, b_out2):
    swish = jax.nn.silu
    row, col = edge_index[0], edge_index[1]
    x_emb = jnp.take(table, x, axis=0)
    posP = jnp.pad(pos, ((0, 0), (0, HIDDEN - 3)))
    prP, pcP = _edge_gather(posP, row, col)
    rel3 = prP[:, :3] - pcP[:, :3]
    t = jnp.maximum(jnp.linalg.norm(rel3, axis=-1, keepdims=True), 1e-6)
    relnP = (prP - pcP) / t
    eau = _eau_kernel(relnP, edge_weights)
    esum11 = jax.ops.segment_sum(eau, col, num_segments=N)
    node_attr = esum11[:, :SH] / jnp.maximum(esum11[:, SH + 1], 1.0)[:, None]
    h = swish(jnp.concatenate([x_emb, node_attr], axis=-1) @ W_in + b_in)
    counts = jax.ops.segment_sum(jnp.ones((N,), jnp.float32), batch, num_segments=G)
    for l in range(DEPTH):
        hr, hc = _edge_gather(h, row, col)
        m = _msg_mlp(hr, hc, eau, msg_W1[l], msg_b1[l], msg_W2[l], msg_b2[l])
        agg = jax.ops.segment_sum(m, col, num_segments=N)
        h = _upd_mlp(h, agg, node_attr, upd_W1[l], upd_b1[l], upd_W2[l], upd_b2[l])
        mean = jax.ops.segment_sum(h, batch, num_segments=G) / counts[:, None]
        var = jax.ops.segment_sum(h * h, batch, num_segments=G) / counts[:, None] - mean ** 2
        h = (h - mean[batch]) / jnp.sqrt(jnp.maximum(var[batch], 0.0) + 1e-5)
    pooled = jax.ops.segment_sum(h, batch, num_segments=G) / counts[:, None]
    out = swish(pooled @ W_out1 + b_out1) @ W_out2 + b_out2
    return out
